# baseline (speedup 1.0000x reference)
import jax
import jax.numpy as jnp
from jax import lax
from jax.experimental import pallas as pl
from jax.experimental.pallas import tpu as pltpu
from jax.experimental.pallas import tpu_sc as plsc

NUM_EMB = 1000000
DIM = 32
BATCH = 4096
HIST = 200

NC = 2
NS = 16
NW = NC * NS

BPW = BATCH // NW
HPC = 4
CTOK = HPC * BPW
NCH = HIST // HPC
NBUF = 2
ROUNDS = NCH // NBUF
OUT_WORDS = BATCH * HIST * DIM
OCH = HPC * DIM * BPW


def _emb_body(tok_hbm, w_hbm, out_hbm, idx_v, g0, g1, o0, o1,
              gs0, gs1, os0, os1):
    gbufs = (g0, g1)
    obufs = (o0, o1)
    gsems = (gs0, gs1)
    osems = (os0, os1)

    cid = lax.axis_index("c")
    sid = lax.axis_index("s")
    w = sid * NC + cid

    pltpu.sync_copy(tok_hbm.at[w], idx_v)

    iota128 = lax.iota(jnp.int32, 16) * BPW
    idx8 = [iota128 + d for d in range(8)]

    def fire_gather(ch, b):
        pltpu.async_copy(w_hbm.at[idx_v.at[ch]], gbufs[b], gsems[b])

    def drain_gather(ch, b):
        pltpu.make_async_copy(w_hbm.at[idx_v.at[ch]], gbufs[b],
                              gsems[b]).wait()

    def fire_writes(ch, b):
        h0 = ch * HPC
        for hh in range(HPC):
            for g in range(4):
                base = ((h0 + hh) * 1024 + g * 256 + w * 8) * BPW
                src = obufs[b].at[pl.ds(hh * DIM * BPW + g * 8 * BPW,
                                        8 * BPW)]
                pltpu.async_copy(src, out_hbm.at[pl.ds(base, 8 * BPW)],
                                 osems[b])

    def drain_writes(b):
        pltpu.make_async_copy(obufs[b], out_hbm.at[pl.ds(0, OCH)],
                              osems[b]).wait()

    def extract(b):
        for hh in range(HPC):
            for bm0 in range(0, BPW, 8):
                pend = []
                for bm in range(bm0, bm0 + 8):
                    for half in range(DIM // 16):
                        v = gbufs[b][hh * BPW + bm, pl.ds(half * 16, 16)]
                        pend.append((half, bm, v))
                for half, bm, v in pend:
                    off = hh * DIM * BPW + half * 16 * BPW + (bm & ~7)
                    view = obufs[b].at[pl.ds(off, 15 * BPW + 8)]
                    plsc.store_scatter(view, [idx8[bm & 7]], v)

    for b in range(NBUF):
        fire_gather(b, b)

    def round_body(r, carry):
        c0 = r * NBUF
        for b in range(NBUF):
            ch = c0 + b
            drain_gather(ch, b)

            @pl.when(r > 0)
            def _():
                drain_writes(b)

            extract(b)
            fire_writes(ch, b)

            @pl.when(ch + NBUF < NCH)
            def _():
                fire_gather(ch + NBUF, b)
        return carry

    lax.fori_loop(0, ROUNDS, round_body, 0)
    for b in range(NBUF):
        drain_writes(b)


def kernel(tokens_ids, weights):
    tokA = tokens_ids.T.reshape(HIST, NW, BPW).transpose(1, 0, 2)
    tokA = tokA.reshape(NW, NCH, CTOK)
    out = pl.kernel(
        _emb_body,
        out_type=jax.ShapeDtypeStruct((OUT_WORDS,), jnp.float32),
        mesh=plsc.VectorSubcoreMesh(
            core_axis_name="c", subcore_axis_name="s",
            num_cores=NC, num_subcores=NS,
        ),
        scratch_types=(
            [pltpu.VMEM((NCH, CTOK), jnp.int32)]
            + [pltpu.VMEM((CTOK, DIM), jnp.float32) for _ in range(NBUF)]
            + [pltpu.VMEM((OCH,), jnp.float32) for _ in range(NBUF)]
            + [pltpu.SemaphoreType.DMA for _ in range(2 * NBUF)]
        ),
        compiler_params=pltpu.CompilerParams(use_tc_tiling_on_sc=False,
                                             needs_layout_passes=False),
    )(tokA, weights)
    r = out.reshape(HIST, 4, NW, 8, BPW).transpose(2, 4, 0, 1, 3)
    return r.reshape(BATCH, HIST, DIM)


# speedup vs baseline: 1.3197x; 1.3197x over previous
import jax
import jax.numpy as jnp
from jax import lax
from jax.experimental import pallas as pl
from jax.experimental.pallas import tpu as pltpu
from jax.experimental.pallas import tpu_sc as plsc

NUM_EMB = 1000000
DIM = 32
BATCH = 4096
HIST = 200

NC = 2
NS = 16
NW = NC * NS

BPW = BATCH // NW
HPC = 4
CTOK = HPC * BPW
NCH = HIST // HPC
NBUF = 2
ROUNDS = NCH // NBUF
OUT_WORDS = BATCH * HIST * DIM
OCH = HPC * DIM * BPW
OSTR = BPW + 8


def _emb_body(tok_hbm, w_hbm, out_hbm, idx_v, g0, g1, o0, o1,
              gs0, gs1, os0, os1):
    gbufs = (g0, g1)
    obufs = (o0, o1)
    gsems = (gs0, gs1)
    osems = (os0, os1)

    cid = lax.axis_index("c")
    sid = lax.axis_index("s")
    w = sid * NC + cid

    pltpu.sync_copy(tok_hbm.at[w], idx_v)

    iota128 = lax.iota(jnp.int32, 16) * OSTR
    idx8 = [iota128 + d for d in range(8)]

    def fire_gather(ch, b):
        pltpu.async_copy(w_hbm.at[idx_v.at[ch]], gbufs[b], gsems[b])

    def drain_gather(ch, b):
        pltpu.make_async_copy(w_hbm.at[idx_v.at[ch]], gbufs[b],
                              gsems[b]).wait()

    def fire_writes(ch, b):
        h0 = ch * HPC
        for hh in range(HPC):
            for c in range(DIM):
                base = ((h0 + hh) * 1024 + (c // 8) * 256 + w * 8
                        + (c % 8)) * BPW
                src = obufs[b].at[pl.ds((hh * DIM + c) * OSTR, BPW)]
                pltpu.async_copy(src, out_hbm.at[pl.ds(base, BPW)],
                                 osems[b])

    def drain_writes(b):
        pltpu.make_async_copy(obufs[b].at[pl.ds(0, OCH)],
                              out_hbm.at[pl.ds(0, OCH)],
                              osems[b]).wait()

    def extract(b):
        for hh in range(HPC):
            for bm0 in range(0, BPW, 8):
                pend = []
                for bm in range(bm0, bm0 + 8):
                    for half in range(DIM // 16):
                        v = gbufs[b][hh * BPW + bm, pl.ds(half * 16, 16)]
                        pend.append((half, bm, v))
                for half, bm, v in pend:
                    off = hh * DIM * OSTR + half * 16 * OSTR + (bm & ~7)
                    view = obufs[b].at[pl.ds(off, 15 * OSTR + 8)]
                    plsc.store_scatter(view, [idx8[bm & 7]], v)

    for b in range(NBUF):
        fire_gather(b, b)

    def round_body(r, carry):
        c0 = r * NBUF
        for b in range(NBUF):
            ch = c0 + b
            drain_gather(ch, b)

            @pl.when(r > 0)
            def _():
                drain_writes(b)

            extract(b)
            fire_writes(ch, b)

            @pl.when(ch + NBUF < NCH)
            def _():
                fire_gather(ch + NBUF, b)
        return carry

    lax.fori_loop(0, ROUNDS, round_body, 0)
    for b in range(NBUF):
        drain_writes(b)


def kernel(tokens_ids, weights):
    tokA = tokens_ids.T.reshape(HIST, NW, BPW).transpose(1, 0, 2)
    tokA = tokA.reshape(NW, NCH, CTOK)
    out = pl.kernel(
        _emb_body,
        out_type=jax.ShapeDtypeStruct((OUT_WORDS,), jnp.float32),
        mesh=plsc.VectorSubcoreMesh(
            core_axis_name="c", subcore_axis_name="s",
            num_cores=NC, num_subcores=NS,
        ),
        scratch_types=(
            [pltpu.VMEM((NCH, CTOK), jnp.int32)]
            + [pltpu.VMEM((CTOK, DIM), jnp.float32) for _ in range(NBUF)]
            + [pltpu.VMEM((HPC * DIM * OSTR,), jnp.float32)
               for _ in range(NBUF)]
            + [pltpu.SemaphoreType.DMA for _ in range(2 * NBUF)]
        ),
        compiler_params=pltpu.CompilerParams(use_tc_tiling_on_sc=False,
                                             needs_layout_passes=False),
    )(tokA, weights)
    r = out.reshape(HIST, 4, NW, 8, BPW).transpose(2, 4, 0, 1, 3)
    return r.reshape(BATCH, HIST, DIM)


# speedup vs baseline: 1.3994x; 1.0604x over previous
import jax
import jax.numpy as jnp
from jax import lax
from jax.experimental import pallas as pl
from jax.experimental.pallas import tpu as pltpu
from jax.experimental.pallas import tpu_sc as plsc

NUM_EMB = 1000000
DIM = 32
BATCH = 4096
HIST = 200

NC = 2
NS = 16
NW = NC * NS

BPW = BATCH // NW
HPC = 2
CTOK = HPC * BPW
NCH = HIST // HPC
NBUF = 2
ROUNDS = NCH // NBUF
OUT_WORDS = BATCH * HIST * DIM
OCH = HPC * DIM * BPW
OSTR = BPW + 8
WROW = 128

PBLK = 2048


def _pad_body(x_ref, o_ref):
    t = x_ref[...].T
    o_ref[...] = jnp.concatenate(
        [t, jnp.zeros((PBLK, WROW - DIM), jnp.float32)], axis=1)


def _emb_body(tok_hbm, w_hbm, out_hbm, idx_v, g0, g1, o0, o1,
              gs0, gs1, os0, os1):
    gbufs = (g0, g1)
    obufs = (o0, o1)
    gsems = (gs0, gs1)
    osems = (os0, os1)

    cid = lax.axis_index("c")
    sid = lax.axis_index("s")
    w = sid * NC + cid

    pltpu.sync_copy(tok_hbm.at[w], idx_v)

    iota128 = lax.iota(jnp.int32, 16) * OSTR
    idx8 = [iota128 + d for d in range(8)]

    def fire_gather(ch, b):
        pltpu.async_copy(w_hbm.at[idx_v.at[ch]], gbufs[b], gsems[b])

    def drain_gather(ch, b):
        pltpu.make_async_copy(w_hbm.at[idx_v.at[ch]], gbufs[b],
                              gsems[b]).wait()

    def fire_writes(ch, b):
        h0 = ch * HPC
        for hh in range(HPC):
            for c in range(DIM):
                base = ((h0 + hh) * 1024 + (c // 8) * 256 + w * 8
                        + (c % 8)) * BPW
                src = obufs[b].at[pl.ds((hh * DIM + c) * OSTR, BPW)]
                pltpu.async_copy(src, out_hbm.at[pl.ds(base, BPW)],
                                 osems[b])

    def drain_writes(b):
        pltpu.make_async_copy(obufs[b].at[pl.ds(0, OCH)],
                              out_hbm.at[pl.ds(0, OCH)],
                              osems[b]).wait()

    def extract(b):
        for hh in range(HPC):
            for bm0 in range(0, BPW, 8):
                pend = []
                for bm in range(bm0, bm0 + 8):
                    for half in range(DIM // 16):
                        v = gbufs[b][hh * BPW + bm, pl.ds(half * 16, 16)]
                        pend.append((half, bm, v))
                for half, bm, v in pend:
                    off = hh * DIM * OSTR + half * 16 * OSTR + (bm & ~7)
                    view = obufs[b].at[pl.ds(off, 15 * OSTR + 8)]
                    plsc.store_scatter(view, [idx8[bm & 7]], v)

    for b in range(NBUF):
        fire_gather(b, b)

    def round_body(r, carry):
        c0 = r * NBUF
        for b in range(NBUF):
            ch = c0 + b
            drain_gather(ch, b)

            @pl.when(r > 0)
            def _():
                drain_writes(b)

            extract(b)
            fire_writes(ch, b)

            @pl.when(ch + NBUF < NCH)
            def _():
                fire_gather(ch + NBUF, b)
        return carry

    lax.fori_loop(0, ROUNDS, round_body, 0)
    for b in range(NBUF):
        drain_writes(b)


def kernel(tokens_ids, weights):
    w128 = pl.pallas_call(
        _pad_body,
        grid=((NUM_EMB + PBLK - 1) // PBLK,),
        in_specs=[pl.BlockSpec((DIM, PBLK), lambda i: (0, i))],
        out_specs=pl.BlockSpec((PBLK, WROW), lambda i: (i, 0)),
        out_shape=jax.ShapeDtypeStruct((NUM_EMB, WROW), jnp.float32),
    )(weights.T)

    tokA = tokens_ids.T.reshape(HIST, NW, BPW).transpose(1, 0, 2)
    tokA = tokA.reshape(NW, NCH, CTOK)
    out = pl.kernel(
        _emb_body,
        out_type=jax.ShapeDtypeStruct((OUT_WORDS,), jnp.float32),
        mesh=plsc.VectorSubcoreMesh(
            core_axis_name="c", subcore_axis_name="s",
            num_cores=NC, num_subcores=NS,
        ),
        scratch_types=(
            [pltpu.VMEM((NCH, CTOK), jnp.int32)]
            + [pltpu.VMEM((CTOK, WROW), jnp.float32) for _ in range(NBUF)]
            + [pltpu.VMEM((HPC * DIM * OSTR,), jnp.float32)
               for _ in range(NBUF)]
            + [pltpu.SemaphoreType.DMA for _ in range(2 * NBUF)]
        ),
        compiler_params=pltpu.CompilerParams(use_tc_tiling_on_sc=False,
                                             needs_layout_passes=False),
    )(tokA, w128)
    r = out.reshape(HIST, 4, NW, 8, BPW).transpose(2, 4, 0, 1, 3)
    return r.reshape(BATCH, HIST, DIM)


# speedup vs baseline: 1.7182x; 1.2278x over previous
import jax
import jax.numpy as jnp
from jax import lax
from jax.experimental import pallas as pl
from jax.experimental.pallas import tpu as pltpu
from jax.experimental.pallas import tpu_sc as plsc

NUM_EMB = 1000000
DIM = 32
BATCH = 4096
HIST = 200

NC = 2
NS = 16
NW = NC * NS

BPW = BATCH // NW
HPC = 2
CTOK = HPC * BPW
NCH = HIST // HPC
NBUF = 2
ROUNDS = NCH // NBUF
OUT_WORDS = BATCH * HIST * DIM
OCH = HPC * DIM * BPW
OSTR = BPW + 8
WROW = 128

PBLK = 4096


def _pad_body(x_ref, o_ref):
    o_ref[:, 0:DIM] = x_ref[...].T


def _emb_body(tok_hbm, w_hbm, out_hbm, idx_v, g0, g1, o0, o1,
              gs0, gs1, os0, os1):
    gbufs = (g0, g1)
    obufs = (o0, o1)
    gsems = (gs0, gs1)
    osems = (os0, os1)

    cid = lax.axis_index("c")
    sid = lax.axis_index("s")
    w = sid * NC + cid

    pltpu.sync_copy(tok_hbm.at[w], idx_v)

    iota128 = lax.iota(jnp.int32, 16) * OSTR
    idx8 = [iota128 + d for d in range(8)]

    def fire_gather(ch, b):
        pltpu.async_copy(w_hbm.at[idx_v.at[ch]], gbufs[b], gsems[b])

    def drain_gather(ch, b):
        pltpu.make_async_copy(w_hbm.at[idx_v.at[ch]], gbufs[b],
                              gsems[b]).wait()

    def fire_writes(ch, b):
        h0 = ch * HPC
        for hh in range(HPC):
            for c in range(DIM):
                base = ((h0 + hh) * 1024 + (c // 8) * 256 + w * 8
                        + (c % 8)) * BPW
                src = obufs[b].at[pl.ds((hh * DIM + c) * OSTR, BPW)]
                pltpu.async_copy(src, out_hbm.at[pl.ds(base, BPW)],
                                 osems[b])

    def drain_writes(b):
        pltpu.make_async_copy(obufs[b].at[pl.ds(0, OCH)],
                              out_hbm.at[pl.ds(0, OCH)],
                              osems[b]).wait()

    def extract(b):
        for hh in range(HPC):
            for bm0 in range(0, BPW, 8):
                pend = []
                for bm in range(bm0, bm0 + 8):
                    for half in range(DIM // 16):
                        v = gbufs[b][hh * BPW + bm, pl.ds(half * 16, 16)]
                        pend.append((half, bm, v))
                for half, bm, v in pend:
                    off = hh * DIM * OSTR + half * 16 * OSTR + (bm & ~7)
                    view = obufs[b].at[pl.ds(off, 15 * OSTR + 8)]
                    plsc.store_scatter(view, [idx8[bm & 7]], v)

    for b in range(NBUF):
        fire_gather(b, b)

    def round_body(r, carry):
        c0 = r * NBUF
        for b in range(NBUF):
            ch = c0 + b
            drain_gather(ch, b)

            @pl.when(r > 0)
            def _():
                drain_writes(b)

            extract(b)
            fire_writes(ch, b)

            @pl.when(ch + NBUF < NCH)
            def _():
                fire_gather(ch + NBUF, b)
        return carry

    lax.fori_loop(0, ROUNDS, round_body, 0)
    for b in range(NBUF):
        drain_writes(b)


def kernel(tokens_ids, weights):
    w128 = pl.pallas_call(
        _pad_body,
        grid=((NUM_EMB + PBLK - 1) // PBLK,),
        in_specs=[pl.BlockSpec((DIM, PBLK), lambda i: (0, i))],
        out_specs=pl.BlockSpec((PBLK, WROW), lambda i: (i, 0)),
        out_shape=jax.ShapeDtypeStruct((NUM_EMB, WROW), jnp.float32),
    )(weights.T)

    tokA = tokens_ids.T.reshape(HIST, NW, BPW).transpose(1, 0, 2)
    tokA = tokA.reshape(NW, NCH, CTOK)
    out = pl.kernel(
        _emb_body,
        out_type=jax.ShapeDtypeStruct((OUT_WORDS,), jnp.float32),
        mesh=plsc.VectorSubcoreMesh(
            core_axis_name="c", subcore_axis_name="s",
            num_cores=NC, num_subcores=NS,
        ),
        scratch_types=(
            [pltpu.VMEM((NCH, CTOK), jnp.int32)]
            + [pltpu.VMEM((CTOK, WROW), jnp.float32) for _ in range(NBUF)]
            + [pltpu.VMEM((HPC * DIM * OSTR,), jnp.float32)
               for _ in range(NBUF)]
            + [pltpu.SemaphoreType.DMA for _ in range(2 * NBUF)]
        ),
        compiler_params=pltpu.CompilerParams(use_tc_tiling_on_sc=False,
                                             needs_layout_passes=False),
    )(tokA, w128)
    r = out.reshape(HIST, 4, NW, 8, BPW).transpose(2, 4, 0, 1, 3)
    return r.reshape(BATCH, HIST, DIM)


# speedup vs baseline: 1.7255x; 1.0043x over previous
import jax
import jax.numpy as jnp
from jax import lax
from jax.experimental import pallas as pl
from jax.experimental.pallas import tpu as pltpu
from jax.experimental.pallas import tpu_sc as plsc

NUM_EMB = 1000000
DIM = 32
BATCH = 4096
HIST = 200

NC = 2
NS = 16
NW = NC * NS

BPW = BATCH // NW
HPC = 2
CTOK = HPC * BPW
NCH = HIST // HPC
NBUF = 2
ROUNDS = NCH // NBUF
OUT_WORDS = BATCH * HIST * DIM
OCH = HPC * DIM * BPW
OSTR = BPW + 8
WROW = 128

PBLK = 4096


def _pad_body(x_ref, o_ref):
    o_ref[:, 0:DIM] = x_ref[...].T


def _emb_body(tok_hbm, w_hbm, out_hbm, idx_v, g0, g1, o0, o1,
              gs0, gs1, os0, os1):
    gbufs = (g0, g1)
    obufs = (o0, o1)
    gsems = (gs0, gs1)
    osems = (os0, os1)

    cid = lax.axis_index("c")
    sid = lax.axis_index("s")
    w = sid * NC + cid

    pltpu.sync_copy(tok_hbm.at[w], idx_v)

    iota128 = lax.iota(jnp.int32, 16) * OSTR
    idx8 = [iota128 + d for d in range(8)]

    def fire_gather(ch, b):
        pltpu.async_copy(w_hbm.at[idx_v.at[ch]], gbufs[b], gsems[b])

    def drain_gather(ch, b):
        pltpu.make_async_copy(w_hbm.at[idx_v.at[ch]], gbufs[b],
                              gsems[b]).wait()

    def fire_writes(ch, b):
        h0 = ch * HPC
        for hh in range(HPC):
            for c in range(DIM):
                base = ((h0 + hh) * 1024 + (c // 8) * 256 + w * 8
                        + (c % 8)) * BPW
                src = obufs[b].at[pl.ds((hh * DIM + c) * OSTR, BPW)]
                pltpu.async_copy(src, out_hbm.at[pl.ds(base, BPW)],
                                 osems[b])

    def drain_writes(b):
        pltpu.make_async_copy(obufs[b].at[pl.ds(0, OCH)],
                              out_hbm.at[pl.ds(0, OCH)],
                              osems[b]).wait()

    def extract(b):
        for hh in range(HPC):
            for bm0 in range(0, BPW, 16):
                pend = []
                for bm in range(bm0, bm0 + 16):
                    for half in range(DIM // 16):
                        v = gbufs[b][hh * BPW + bm, pl.ds(half * 16, 16)]
                        pend.append((half, bm, v))
                for half, bm, v in pend:
                    off = hh * DIM * OSTR + half * 16 * OSTR + (bm & ~7)
                    view = obufs[b].at[pl.ds(off, 15 * OSTR + 8)]
                    plsc.store_scatter(view, [idx8[bm & 7]], v)

    for b in range(NBUF):
        fire_gather(b, b)

    def round_body(r, carry):
        c0 = r * NBUF
        for b in range(NBUF):
            ch = c0 + b
            drain_gather(ch, b)

            @pl.when(r > 0)
            def _():
                drain_writes(b)

            extract(b)
            fire_writes(ch, b)

            @pl.when(ch + NBUF < NCH)
            def _():
                fire_gather(ch + NBUF, b)
        return carry

    lax.fori_loop(0, ROUNDS, round_body, 0)
    for b in range(NBUF):
        drain_writes(b)


def kernel(tokens_ids, weights):
    w128 = pl.pallas_call(
        _pad_body,
        grid=((NUM_EMB + PBLK - 1) // PBLK,),
        in_specs=[pl.BlockSpec((DIM, PBLK), lambda i: (0, i))],
        out_specs=pl.BlockSpec((PBLK, WROW), lambda i: (i, 0)),
        out_shape=jax.ShapeDtypeStruct((NUM_EMB, WROW), jnp.float32),
    )(weights.T)

    tokA = tokens_ids.T.reshape(HIST, NW, BPW).transpose(1, 0, 2)
    tokA = tokA.reshape(NW, NCH, CTOK)
    out = pl.kernel(
        _emb_body,
        out_type=jax.ShapeDtypeStruct((OUT_WORDS,), jnp.float32),
        mesh=plsc.VectorSubcoreMesh(
            core_axis_name="c", subcore_axis_name="s",
            num_cores=NC, num_subcores=NS,
        ),
        scratch_types=(
            [pltpu.VMEM((NCH, CTOK), jnp.int32)]
            + [pltpu.VMEM((CTOK, WROW), jnp.float32) for _ in range(NBUF)]
            + [pltpu.VMEM((HPC * DIM * OSTR,), jnp.float32)
               for _ in range(NBUF)]
            + [pltpu.SemaphoreType.DMA for _ in range(2 * NBUF)]
        ),
        compiler_params=pltpu.CompilerParams(use_tc_tiling_on_sc=False,
                                             needs_layout_passes=False),
    )(tokA, w128)
    r = out.reshape(HIST, 4, NW, 8, BPW).transpose(2, 4, 0, 1, 3)
    return r.reshape(BATCH, HIST, DIM)


# speedup vs baseline: 1.9636x; 1.1380x over previous
import jax
import jax.numpy as jnp
from jax import lax
from jax.experimental import pallas as pl
from jax.experimental.pallas import tpu as pltpu
from jax.experimental.pallas import tpu_sc as plsc

NUM_EMB = 1000000
DIM = 32
BATCH = 4096
HIST = 200

NC = 2
NS = 16
NW = NC * NS

BPW = BATCH // NW
HPC = 2
CTOK = HPC * BPW
NCH = HIST // HPC
NBUF = 2
ROUNDS = NCH // NBUF
OUT_WORDS = BATCH * HIST * DIM
OCH = HPC * DIM * BPW
OSTR = BPW + 8
WROW = 128

PBLK = 8192


def _pad_body(x_ref, o_ref):
    o_ref[:, 0:DIM] = x_ref[...].T


def _emb_body(tok_hbm, w_hbm, out_hbm, idx_v, g0, g1, o0, o1,
              gs0, gs1, os0, os1):
    gbufs = (g0, g1)
    obufs = (o0, o1)
    gsems = (gs0, gs1)
    osems = (os0, os1)

    cid = lax.axis_index("c")
    sid = lax.axis_index("s")
    w = sid * NC + cid

    pltpu.sync_copy(tok_hbm.at[w], idx_v)

    iota128 = lax.iota(jnp.int32, 16) * OSTR
    idx8 = [iota128 + d for d in range(8)]

    def fire_gather(ch, b):
        pltpu.async_copy(w_hbm.at[idx_v.at[ch]], gbufs[b], gsems[b])

    def drain_gather(ch, b):
        pltpu.make_async_copy(w_hbm.at[idx_v.at[ch]], gbufs[b],
                              gsems[b]).wait()

    def fire_writes(ch, b):
        h0 = ch * HPC
        for hh in range(HPC):
            for c in range(DIM):
                base = ((h0 + hh) * 1024 + (c // 8) * 256 + w * 8
                        + (c % 8)) * BPW
                src = obufs[b].at[pl.ds((hh * DIM + c) * OSTR, BPW)]
                pltpu.async_copy(src, out_hbm.at[pl.ds(base, BPW)],
                                 osems[b])

    def drain_writes(b):
        pltpu.make_async_copy(obufs[b].at[pl.ds(0, OCH)],
                              out_hbm.at[pl.ds(0, OCH)],
                              osems[b]).wait()

    def extract(b):
        for hh in range(HPC):
            for bm0 in range(0, BPW, 16):
                pend = []
                for bm in range(bm0, bm0 + 16):
                    for half in range(DIM // 16):
                        v = gbufs[b][hh * BPW + bm, pl.ds(half * 16, 16)]
                        pend.append((half, bm, v))
                for half, bm, v in pend:
                    off = hh * DIM * OSTR + half * 16 * OSTR + (bm & ~7)
                    view = obufs[b].at[pl.ds(off, 15 * OSTR + 8)]
                    plsc.store_scatter(view, [idx8[bm & 7]], v)

    for b in range(NBUF):
        fire_gather(b, b)

    def round_body(r, carry):
        c0 = r * NBUF
        for b in range(NBUF):
            ch = c0 + b
            drain_gather(ch, b)

            @pl.when(r > 0)
            def _():
                drain_writes(b)

            extract(b)
            fire_writes(ch, b)

            @pl.when(ch + NBUF < NCH)
            def _():
                fire_gather(ch + NBUF, b)
        return carry

    lax.fori_loop(0, ROUNDS, round_body, 0)
    for b in range(NBUF):
        drain_writes(b)


def kernel(tokens_ids, weights):
    w128 = pl.pallas_call(
        _pad_body,
        grid=((NUM_EMB + PBLK - 1) // PBLK,),
        in_specs=[pl.BlockSpec((DIM, PBLK), lambda i: (0, i))],
        out_specs=pl.BlockSpec((PBLK, WROW), lambda i: (i, 0)),
        out_shape=jax.ShapeDtypeStruct((NUM_EMB, WROW), jnp.float32),
    )(weights.T)

    tokA = tokens_ids.T.reshape(HIST, NW, BPW).transpose(1, 0, 2)
    tokA = tokA.reshape(NW, NCH, CTOK)
    out = pl.kernel(
        _emb_body,
        out_type=jax.ShapeDtypeStruct((OUT_WORDS,), jnp.float32),
        mesh=plsc.VectorSubcoreMesh(
            core_axis_name="c", subcore_axis_name="s",
            num_cores=NC, num_subcores=NS,
        ),
        scratch_types=(
            [pltpu.VMEM((NCH, CTOK), jnp.int32)]
            + [pltpu.VMEM((CTOK, WROW), jnp.float32) for _ in range(NBUF)]
            + [pltpu.VMEM((HPC * DIM * OSTR,), jnp.float32)
               for _ in range(NBUF)]
            + [pltpu.SemaphoreType.DMA for _ in range(2 * NBUF)]
        ),
        compiler_params=pltpu.CompilerParams(use_tc_tiling_on_sc=False,
                                             needs_layout_passes=False),
    )(tokA, w128)
    r = out.reshape(HIST, 4, NW, 8, BPW).transpose(2, 4, 0, 1, 3)
    return r.reshape(BATCH, HIST, DIM)


# speedup vs baseline: 2.1035x; 1.0712x over previous
import jax
import jax.numpy as jnp
from jax import lax
from jax.experimental import pallas as pl
from jax.experimental.pallas import tpu as pltpu
from jax.experimental.pallas import tpu_sc as plsc

NUM_EMB = 1000000
DIM = 32
BATCH = 4096
HIST = 200

NC = 2
NS = 16
NW = NC * NS

BPW = BATCH // NW
HPC = 2
CTOK = HPC * BPW
NCH = HIST // HPC
NBUF = 2
ROUNDS = NCH // NBUF
OUT_WORDS = BATCH * HIST * DIM
OCH = HPC * DIM * BPW
OSTR = BPW + 8
WROW = 128

PBLK = 16384


def _pad_body(x_ref, o_ref):
    o_ref[:, 0:DIM] = x_ref[...].T


def _emb_body(tok_hbm, w_hbm, out_hbm, idx_v, g0, g1, o0, o1,
              gs0, gs1, os0, os1):
    gbufs = (g0, g1)
    obufs = (o0, o1)
    gsems = (gs0, gs1)
    osems = (os0, os1)

    cid = lax.axis_index("c")
    sid = lax.axis_index("s")
    w = sid * NC + cid

    pltpu.sync_copy(tok_hbm.at[w], idx_v)

    iota128 = lax.iota(jnp.int32, 16) * OSTR
    idx8 = [iota128 + d for d in range(8)]

    def fire_gather(ch, b):
        pltpu.async_copy(w_hbm.at[idx_v.at[ch]], gbufs[b], gsems[b])

    def drain_gather(ch, b):
        pltpu.make_async_copy(w_hbm.at[idx_v.at[ch]], gbufs[b],
                              gsems[b]).wait()

    def fire_writes(ch, b):
        h0 = ch * HPC
        for hh in range(HPC):
            for c in range(DIM):
                base = ((h0 + hh) * 1024 + (c // 8) * 256 + w * 8
                        + (c % 8)) * BPW
                src = obufs[b].at[pl.ds((hh * DIM + c) * OSTR, BPW)]
                pltpu.async_copy(src, out_hbm.at[pl.ds(base, BPW)],
                                 osems[b])

    def drain_writes(b):
        pltpu.make_async_copy(obufs[b].at[pl.ds(0, OCH)],
                              out_hbm.at[pl.ds(0, OCH)],
                              osems[b]).wait()

    def extract(b):
        for hh in range(HPC):
            for bm0 in range(0, BPW, 16):
                pend = []
                for bm in range(bm0, bm0 + 16):
                    for half in range(DIM // 16):
                        v = gbufs[b][hh * BPW + bm, pl.ds(half * 16, 16)]
                        pend.append((half, bm, v))
                for half, bm, v in pend:
                    off = hh * DIM * OSTR + half * 16 * OSTR + (bm & ~7)
                    view = obufs[b].at[pl.ds(off, 15 * OSTR + 8)]
                    plsc.store_scatter(view, [idx8[bm & 7]], v)

    for b in range(NBUF):
        fire_gather(b, b)

    def round_body(r, carry):
        c0 = r * NBUF
        for b in range(NBUF):
            ch = c0 + b
            drain_gather(ch, b)

            @pl.when(r > 0)
            def _():
                drain_writes(b)

            extract(b)
            fire_writes(ch, b)

            @pl.when(ch + NBUF < NCH)
            def _():
                fire_gather(ch + NBUF, b)
        return carry

    lax.fori_loop(0, ROUNDS, round_body, 0)
    for b in range(NBUF):
        drain_writes(b)


def kernel(tokens_ids, weights):
    w128 = pl.pallas_call(
        _pad_body,
        grid=((NUM_EMB + PBLK - 1) // PBLK,),
        in_specs=[pl.BlockSpec((DIM, PBLK), lambda i: (0, i))],
        out_specs=pl.BlockSpec((PBLK, WROW), lambda i: (i, 0)),
        out_shape=jax.ShapeDtypeStruct((NUM_EMB, WROW), jnp.float32),
    )(weights.T)

    tokA = tokens_ids.T.reshape(HIST, NW, BPW).transpose(1, 0, 2)
    tokA = tokA.reshape(NW, NCH, CTOK)
    out = pl.kernel(
        _emb_body,
        out_type=jax.ShapeDtypeStruct((OUT_WORDS,), jnp.float32),
        mesh=plsc.VectorSubcoreMesh(
            core_axis_name="c", subcore_axis_name="s",
            num_cores=NC, num_subcores=NS,
        ),
        scratch_types=(
            [pltpu.VMEM((NCH, CTOK), jnp.int32)]
            + [pltpu.VMEM((CTOK, WROW), jnp.float32) for _ in range(NBUF)]
            + [pltpu.VMEM((HPC * DIM * OSTR,), jnp.float32)
               for _ in range(NBUF)]
            + [pltpu.SemaphoreType.DMA for _ in range(2 * NBUF)]
        ),
        compiler_params=pltpu.CompilerParams(use_tc_tiling_on_sc=False,
                                             needs_layout_passes=False),
    )(tokA, w128)
    r = out.reshape(HIST, 4, NW, 8, BPW).transpose(2, 4, 0, 1, 3)
    return r.reshape(BATCH, HIST, DIM)


# speedup vs baseline: 2.1296x; 1.0124x over previous
import jax
import jax.numpy as jnp
from jax import lax
from jax.experimental import pallas as pl
from jax.experimental.pallas import tpu as pltpu
from jax.experimental.pallas import tpu_sc as plsc

NUM_EMB = 1000000
DIM = 32
BATCH = 4096
HIST = 200

NC = 2
NS = 16
NW = NC * NS

BPW = BATCH // NW
HPC = 2
CTOK = HPC * BPW
NCH = HIST // HPC
NBUF = 2
ROUNDS = NCH // NBUF
OUT_WORDS = BATCH * HIST * DIM
OCH = HPC * DIM * BPW
OSTR = BPW + 8
WROW = 128

PBLK = 32768


def _pad_body(x_ref, o_ref):
    o_ref[:, 0:DIM] = x_ref[...].T


def _emb_body(tok_hbm, w_hbm, out_hbm, idx_v, g0, g1, o0, o1,
              gs0, gs1, os0, os1):
    gbufs = (g0, g1)
    obufs = (o0, o1)
    gsems = (gs0, gs1)
    osems = (os0, os1)

    cid = lax.axis_index("c")
    sid = lax.axis_index("s")
    w = sid * NC + cid

    pltpu.sync_copy(tok_hbm.at[w], idx_v)

    iota128 = lax.iota(jnp.int32, 16) * OSTR
    idx8 = [iota128 + d for d in range(8)]

    def fire_gather(ch, b):
        pltpu.async_copy(w_hbm.at[idx_v.at[ch]], gbufs[b], gsems[b])

    def drain_gather(ch, b):
        pltpu.make_async_copy(w_hbm.at[idx_v.at[ch]], gbufs[b],
                              gsems[b]).wait()

    def fire_writes(ch, b):
        h0 = ch * HPC
        for hh in range(HPC):
            for c in range(DIM):
                base = ((h0 + hh) * 1024 + (c // 8) * 256 + w * 8
                        + (c % 8)) * BPW
                src = obufs[b].at[pl.ds((hh * DIM + c) * OSTR, BPW)]
                pltpu.async_copy(src, out_hbm.at[pl.ds(base, BPW)],
                                 osems[b])

    def drain_writes(b):
        pltpu.make_async_copy(obufs[b].at[pl.ds(0, OCH)],
                              out_hbm.at[pl.ds(0, OCH)],
                              osems[b]).wait()

    def extract(b):
        for hh in range(HPC):
            for bm0 in range(0, BPW, 16):
                pend = []
                for bm in range(bm0, bm0 + 16):
                    for half in range(DIM // 16):
                        v = gbufs[b][hh * BPW + bm, pl.ds(half * 16, 16)]
                        pend.append((half, bm, v))
                for half, bm, v in pend:
                    off = hh * DIM * OSTR + half * 16 * OSTR + (bm & ~7)
                    view = obufs[b].at[pl.ds(off, 15 * OSTR + 8)]
                    plsc.store_scatter(view, [idx8[bm & 7]], v)

    for b in range(NBUF):
        fire_gather(b, b)

    def round_body(r, carry):
        c0 = r * NBUF
        for b in range(NBUF):
            ch = c0 + b
            drain_gather(ch, b)

            @pl.when(r > 0)
            def _():
                drain_writes(b)

            extract(b)
            fire_writes(ch, b)

            @pl.when(ch + NBUF < NCH)
            def _():
                fire_gather(ch + NBUF, b)
        return carry

    lax.fori_loop(0, ROUNDS, round_body, 0)
    for b in range(NBUF):
        drain_writes(b)


def kernel(tokens_ids, weights):
    w128 = pl.pallas_call(
        _pad_body,
        grid=((NUM_EMB + PBLK - 1) // PBLK,),
        in_specs=[pl.BlockSpec((DIM, PBLK), lambda i: (0, i))],
        out_specs=pl.BlockSpec((PBLK, WROW), lambda i: (i, 0)),
        out_shape=jax.ShapeDtypeStruct((NUM_EMB, WROW), jnp.float32),
    )(weights.T)

    tokA = tokens_ids.T.reshape(HIST, NW, BPW).transpose(1, 0, 2)
    tokA = tokA.reshape(NW, NCH, CTOK)
    out = pl.kernel(
        _emb_body,
        out_type=jax.ShapeDtypeStruct((OUT_WORDS,), jnp.float32),
        mesh=plsc.VectorSubcoreMesh(
            core_axis_name="c", subcore_axis_name="s",
            num_cores=NC, num_subcores=NS,
        ),
        scratch_types=(
            [pltpu.VMEM((NCH, CTOK), jnp.int32)]
            + [pltpu.VMEM((CTOK, WROW), jnp.float32) for _ in range(NBUF)]
            + [pltpu.VMEM((HPC * DIM * OSTR,), jnp.float32)
               for _ in range(NBUF)]
            + [pltpu.SemaphoreType.DMA for _ in range(2 * NBUF)]
        ),
        compiler_params=pltpu.CompilerParams(use_tc_tiling_on_sc=False,
                                             needs_layout_passes=False),
    )(tokA, w128)
    r = out.reshape(HIST, 4, NW, 8, BPW).transpose(2, 4, 0, 1, 3)
    return r.reshape(BATCH, HIST, DIM)
